# baseline (speedup 1.0000x reference)
import jax
import jax.numpy as jnp
from jax import lax
from jax.experimental import pallas as pl
from jax.experimental.pallas import tpu as pltpu
from jax.experimental.pallas import tpu_sc as plsc

_N = 100000
_H = 128
_NB = 100000
_NC = 100000

_B = 128
_HB = _H // 16
_RPW = 3200
_CPW = _RPW // _B
_LAST_FULL = 6
_TAIL = 32


def _sc_body(feat_hbm, tail_hbm, btab_hbm, ctab_hbm, wb_hbm, wc_hbm, out_hbm,
             slab_a, slab_b, idx0_a, idx0_b, idx1_a, idx1_b,
             idxc_a, idxc_b, sb_a, sb_b, sc_a, sc_b,
             wb_v, wc_v, out_a, out_b,
             sem_in_a, sem_in_b, sem_g_a, sem_g_b, sem_out_a, sem_out_b):
    wid = lax.axis_index("s") * 2 + lax.axis_index("c")
    row0 = wid * _RPW

    pltpu.sync_copy(wb_hbm, wb_v)
    pltpu.sync_copy(wc_hbm, wc_v)
    wbs = [wb_v[pl.ds(h * 16, 16)] for h in range(_HB)]
    wcs = [wc_v[pl.ds(h * 16, 16)] for h in range(_HB)]

    bufs = (
        (slab_a, idx0_a, idx1_a, idxc_a, sb_a, sc_a, out_a,
         sem_in_a, sem_g_a, sem_out_a),
        (slab_b, idx0_b, idx1_b, idxc_b, sb_b, sc_b, out_b,
         sem_in_b, sem_g_b, sem_out_b),
    )

    def fire_slab(j, p):
        slab, sem = bufs[p][0], bufs[p][7]
        base = pl.multiple_of(row0 + j * _B, _B)
        pltpu.async_copy(feat_hbm.at[:, pl.ds(base, _B)], slab, sem)

    def wait_slab(p):
        slab, sem = bufs[p][0], bufs[p][7]
        pltpu.make_async_copy(feat_hbm.at[:, pl.ds(0, _B)], slab, sem).wait()

    def compute(p, nrows=_B):
        slab, idx0, idx1, idxc, sb, sc = bufs[p][:6]
        for g in range(nrows // 16):
            def col(c):
                return slab[c, pl.ds(g * 16, 16)]

            idx0[pl.ds(g * 16, 16)] = jnp.clip(
                (col(0) * _NB).astype(jnp.int32), 0, _NB - 1)
            idx1[pl.ds(g * 16, 16)] = jnp.clip(
                (col(1) * _NB).astype(jnp.int32), 0, _NB - 1)
            idxc[pl.ds(g * 16, 16)] = jnp.clip(
                (col(11) * _NC).astype(jnp.int32), 0, _NC - 1)

            s = col(2)
            for c in range(3, 10):
                s = s + col(c)
            sb[pl.ds(g * 16, 16)] = s
            sc[pl.ds(g * 16, 16)] = col(10)

    def dense_init(p, nrows=_B):
        sb, sc, out = bufs[p][4], bufs[p][5], bufs[p][6]

        def rows16(g, _):
            base = pl.multiple_of(g * 16, 16)
            sbv = sb[pl.ds(base, 16)]
            scv = sc[pl.ds(base, 16)]
            for k in range(16):
                vb = jnp.full((16,), sbv[k])
                vc = jnp.full((16,), scv[k])
                r = base + k
                for h in range(_HB):
                    out[r, pl.ds(h * 16, 16)] = vb * wbs[h] + vc * wcs[h]
            return 0

        lax.fori_loop(0, nrows // 16, rows16, 0)

    def fire_gathers(p, nrows=_B):
        idx0, idx1, idxc, out, sem = (bufs[p][1], bufs[p][2], bufs[p][3],
                                      bufs[p][6], bufs[p][8])
        dst = out.at[pl.ds(0, nrows)]
        return (
            pltpu.async_copy(btab_hbm.at[idx0.at[pl.ds(0, nrows)]], dst, sem,
                             add=True),
            pltpu.async_copy(btab_hbm.at[idx1.at[pl.ds(0, nrows)]], dst, sem,
                             add=True),
            pltpu.async_copy(ctab_hbm.at[idxc.at[pl.ds(0, nrows)]], dst, sem,
                             add=True),
        )

    def fire_scatter(j, p):
        out, sem = bufs[p][6], bufs[p][9]
        base = pl.multiple_of(row0 + j * _B, _B)
        pltpu.async_copy(out, out_hbm.at[0, pl.ds(base, _B)], sem)

    def wait_scatter(p):
        out, sem = bufs[p][6], bufs[p][9]
        pltpu.make_async_copy(out, out_hbm.at[0, pl.ds(0, _B)], sem).wait()

    @pl.when(wid < 31)
    def _pipelined():
        fire_slab(0, 0)
        fire_scatter(1, 1)
        wait_slab(0)
        fire_slab(1, 1)
        compute(0)
        dense_init(0)

        def half(j_g, p, guard_slab):
            q = 1 - p
            cps = fire_gathers(p)
            wait_slab(q)

            if guard_slab:
                @pl.when(j_g + 2 < _CPW)
                def _():
                    fire_slab(j_g + 2, p)
            else:
                fire_slab(j_g + 2, p)

            compute(q)
            wait_scatter(q)
            dense_init(q)
            for cp in cps:
                cp.wait()
            fire_scatter(j_g, p)

        def pair(i, carry):
            half(2 * i, 0, False)
            half(2 * i + 1, 1, True)
            return carry

        lax.fori_loop(0, (_CPW - 1) // 2, pair, 0)

        cps = fire_gathers(0)
        for cp in cps:
            cp.wait()
        fire_scatter(_CPW - 1, 0)
        wait_scatter(1)
        wait_scatter(0)

    @pl.when(wid == 31)
    def _sequential():
        slab, out, sem_in = bufs[0][0], bufs[0][6], bufs[0][7]

        def chunk(j, carry):
            base = pl.multiple_of(row0 + j * _B, _B)
            pltpu.async_copy(feat_hbm.at[:, pl.ds(base, _B)], slab,
                             sem_in).wait()
            compute(0)
            dense_init(0)
            for cp in fire_gathers(0):
                cp.wait()
            pltpu.sync_copy(out, out_hbm.at[0, pl.ds(base, _B)])
            return carry

        lax.fori_loop(0, _LAST_FULL, chunk, 0)

        pltpu.async_copy(tail_hbm, slab, sem_in).wait()
        compute(0)
        dense_init(0)
        for cp in fire_gathers(0):
            cp.wait()
        pltpu.sync_copy(out, out_hbm.at[0, pl.ds(_N - _B, _B)])


def kernel(node_feat, basic_table, contact_table, W_basic, W_contact):
    feat_t = node_feat.T
    tail_t = node_feat[_N - _B:].T
    wb = W_basic.reshape(_H)
    wc = W_contact.reshape(_H)
    run = pl.kernel(
        _sc_body,
        out_type=jax.ShapeDtypeStruct((1, _N, _H), jnp.float32),
        mesh=plsc.VectorSubcoreMesh(core_axis_name="c", subcore_axis_name="s"),
        compiler_params=pltpu.CompilerParams(needs_layout_passes=False),
        scratch_types=[
            pltpu.VMEM((12, _B), jnp.float32),
            pltpu.VMEM((12, _B), jnp.float32),
            pltpu.VMEM((_B,), jnp.int32),
            pltpu.VMEM((_B,), jnp.int32),
            pltpu.VMEM((_B,), jnp.int32),
            pltpu.VMEM((_B,), jnp.int32),
            pltpu.VMEM((_B,), jnp.int32),
            pltpu.VMEM((_B,), jnp.int32),
            pltpu.VMEM((_B,), jnp.float32),
            pltpu.VMEM((_B,), jnp.float32),
            pltpu.VMEM((_B,), jnp.float32),
            pltpu.VMEM((_B,), jnp.float32),
            pltpu.VMEM((_H,), jnp.float32),
            pltpu.VMEM((_H,), jnp.float32),
            pltpu.VMEM((_B, _H), jnp.float32),
            pltpu.VMEM((_B, _H), jnp.float32),
            pltpu.SemaphoreType.DMA,
            pltpu.SemaphoreType.DMA,
            pltpu.SemaphoreType.DMA,
            pltpu.SemaphoreType.DMA,
            pltpu.SemaphoreType.DMA,
            pltpu.SemaphoreType.DMA,
        ],
    )
    return run(feat_t, tail_t, basic_table, contact_table, wb, wc)


# speedup vs baseline: 1.0173x; 1.0173x over previous
import jax
import jax.numpy as jnp
from jax import lax
from jax.experimental import pallas as pl
from jax.experimental.pallas import tpu as pltpu
from jax.experimental.pallas import tpu_sc as plsc

_N = 100000
_H = 128
_NB = 100000
_NC = 100000

_B = 128
_HB = _H // 16
_RPW = 3200
_CPW = _RPW // _B
_LAST_FULL = 6
_TAIL = 32


def _sc_body(feat_hbm, tail_hbm, btab_hbm, ctab_hbm, wb_hbm, wc_hbm, out_hbm,
             slab_a, slab_b, idx0_a, idx0_b, idx1_a, idx1_b,
             idxc_a, idxc_b, sb_a, sb_b, sc_a, sc_b,
             wb_v, wc_v, out_a, out_b,
             sem_in_a, sem_in_b, sem_g_a, sem_g_b, sem_out_a, sem_out_b):
    wid = lax.axis_index("s") * 2 + lax.axis_index("c")
    row0 = wid * _RPW

    pltpu.sync_copy(wb_hbm, wb_v)
    pltpu.sync_copy(wc_hbm, wc_v)
    wbs = [wb_v[pl.ds(h * 16, 16)] for h in range(_HB)]
    wcs = [wc_v[pl.ds(h * 16, 16)] for h in range(_HB)]

    bufs = (
        (slab_a, idx0_a, idx1_a, idxc_a, sb_a, sc_a, out_a,
         sem_in_a, sem_g_a, sem_out_a),
        (slab_b, idx0_b, idx1_b, idxc_b, sb_b, sc_b, out_b,
         sem_in_b, sem_g_b, sem_out_b),
    )

    def fire_slab(j, p):
        slab, sem = bufs[p][0], bufs[p][7]
        base = pl.multiple_of(row0 + j * _B, _B)
        pltpu.async_copy(feat_hbm.at[:, pl.ds(base, _B)], slab, sem)

    def wait_slab(p):
        slab, sem = bufs[p][0], bufs[p][7]
        pltpu.make_async_copy(feat_hbm.at[:, pl.ds(0, _B)], slab, sem).wait()

    def compute(p, nrows=_B):
        slab, idx0, idx1, idxc, sb, sc = bufs[p][:6]
        for g in range(nrows // 16):
            def col(c):
                return slab[c, pl.ds(g * 16, 16)]

            idx0[pl.ds(g * 16, 16)] = jnp.clip(
                (col(0) * _NB).astype(jnp.int32), 0, _NB - 1)
            idx1[pl.ds(g * 16, 16)] = jnp.clip(
                (col(1) * _NB).astype(jnp.int32), 0, _NB - 1)
            idxc[pl.ds(g * 16, 16)] = jnp.clip(
                (col(11) * _NC).astype(jnp.int32), 0, _NC - 1)

            s = col(2)
            for c in range(3, 10):
                s = s + col(c)
            sb[pl.ds(g * 16, 16)] = s
            sc[pl.ds(g * 16, 16)] = col(10)

    def dense_init(p, nrows=_B):
        sb, sc, out = bufs[p][4], bufs[p][5], bufs[p][6]

        def rows16(g, _):
            base = pl.multiple_of(g * 16, 16)
            sbv = sb[pl.ds(base, 16)]
            scv = sc[pl.ds(base, 16)]
            for k in range(16):
                vb = jnp.full((16,), sbv[k])
                vc = jnp.full((16,), scv[k])
                r = base + k
                for h in range(_HB):
                    out[r, pl.ds(h * 16, 16)] = vb * wbs[h] + vc * wcs[h]
            return 0

        lax.fori_loop(0, nrows // 16, rows16, 0)

    def fire_gathers(p, nrows=_B):
        idx0, idx1, idxc, out, sem = (bufs[p][1], bufs[p][2], bufs[p][3],
                                      bufs[p][6], bufs[p][8])
        dst = out.at[pl.ds(0, nrows)]
        return (
            pltpu.async_copy(btab_hbm.at[idx0.at[pl.ds(0, nrows)]], dst, sem,
                             add=True),
            pltpu.async_copy(btab_hbm.at[idx1.at[pl.ds(0, nrows)]], dst, sem,
                             add=True),
            pltpu.async_copy(ctab_hbm.at[idxc.at[pl.ds(0, nrows)]], dst, sem,
                             add=True),
        )

    def fire_scatter(j, p):
        out, sem = bufs[p][6], bufs[p][9]
        base = pl.multiple_of(row0 + j * _B, _B)
        pltpu.async_copy(out, out_hbm.at[0, pl.ds(base, _B)], sem)

    def wait_scatter(p):
        out, sem = bufs[p][6], bufs[p][9]
        pltpu.make_async_copy(out, out_hbm.at[0, pl.ds(0, _B)], sem).wait()

    @pl.when(wid < 31)
    def _pipelined():
        fire_slab(0, 0)
        fire_scatter(1, 1)
        wait_slab(0)
        fire_slab(1, 1)
        compute(0)
        dense_init(0)

        def pair(i, carry):
            j = 2 * i
            cps0 = fire_gathers(0)
            wait_slab(1)
            fire_slab(j + 2, 0)
            compute(1)
            wait_scatter(1)
            dense_init(1)
            cps1 = fire_gathers(1)
            for cp in cps0:
                cp.wait()
            fire_scatter(j, 0)
            wait_slab(0)

            @pl.when(j + 3 < _CPW)
            def _():
                fire_slab(j + 3, 1)

            compute(0)
            wait_scatter(0)
            dense_init(0)
            for cp in cps1:
                cp.wait()
            fire_scatter(j + 1, 1)
            return carry

        lax.fori_loop(0, (_CPW - 1) // 2, pair, 0)

        cps = fire_gathers(0)
        for cp in cps:
            cp.wait()
        fire_scatter(_CPW - 1, 0)
        wait_scatter(1)
        wait_scatter(0)

    @pl.when(wid == 31)
    def _sequential():
        slab, out, sem_in = bufs[0][0], bufs[0][6], bufs[0][7]

        def chunk(j, carry):
            base = pl.multiple_of(row0 + j * _B, _B)
            pltpu.async_copy(feat_hbm.at[:, pl.ds(base, _B)], slab,
                             sem_in).wait()
            compute(0)
            dense_init(0)
            for cp in fire_gathers(0):
                cp.wait()
            pltpu.sync_copy(out, out_hbm.at[0, pl.ds(base, _B)])
            return carry

        lax.fori_loop(0, _LAST_FULL, chunk, 0)

        pltpu.async_copy(tail_hbm, slab, sem_in).wait()
        compute(0)
        dense_init(0)
        for cp in fire_gathers(0):
            cp.wait()
        pltpu.sync_copy(out, out_hbm.at[0, pl.ds(_N - _B, _B)])


def kernel(node_feat, basic_table, contact_table, W_basic, W_contact):
    feat_t = node_feat.T
    tail_t = node_feat[_N - _B:].T
    wb = W_basic.reshape(_H)
    wc = W_contact.reshape(_H)
    run = pl.kernel(
        _sc_body,
        out_type=jax.ShapeDtypeStruct((1, _N, _H), jnp.float32),
        mesh=plsc.VectorSubcoreMesh(core_axis_name="c", subcore_axis_name="s"),
        compiler_params=pltpu.CompilerParams(needs_layout_passes=False),
        scratch_types=[
            pltpu.VMEM((12, _B), jnp.float32),
            pltpu.VMEM((12, _B), jnp.float32),
            pltpu.VMEM((_B,), jnp.int32),
            pltpu.VMEM((_B,), jnp.int32),
            pltpu.VMEM((_B,), jnp.int32),
            pltpu.VMEM((_B,), jnp.int32),
            pltpu.VMEM((_B,), jnp.int32),
            pltpu.VMEM((_B,), jnp.int32),
            pltpu.VMEM((_B,), jnp.float32),
            pltpu.VMEM((_B,), jnp.float32),
            pltpu.VMEM((_B,), jnp.float32),
            pltpu.VMEM((_B,), jnp.float32),
            pltpu.VMEM((_H,), jnp.float32),
            pltpu.VMEM((_H,), jnp.float32),
            pltpu.VMEM((_B, _H), jnp.float32),
            pltpu.VMEM((_B, _H), jnp.float32),
            pltpu.SemaphoreType.DMA,
            pltpu.SemaphoreType.DMA,
            pltpu.SemaphoreType.DMA,
            pltpu.SemaphoreType.DMA,
            pltpu.SemaphoreType.DMA,
            pltpu.SemaphoreType.DMA,
        ],
    )
    return run(feat_t, tail_t, basic_table, contact_table, wb, wc)
